# Initial kernel scaffold; baseline (speedup 1.0000x reference)
import jax
import jax.numpy as jnp
from jax.experimental import pallas as pl


def kernel(cls_p3, cntr_p3, regr_p3, cls_p4, cntr_p4, regr_p4, cls_p5, cntr_p5, regr_p5, cls_p6, cntr_p6, regr_p6, cls_p7, cntr_p7, regr_p7):
    raise NotImplementedError("write your pallas kernel here")


# speedup vs baseline: 1.2314x; 1.2314x over previous
import functools

import jax
import jax.numpy as jnp
from jax import lax
from jax.experimental import pallas as pl
from jax.experimental.pallas import tpu as pltpu
from jax.experimental.pallas import tpu_sc as plsc

B = 4
NUM_CLASSES = 80
IMG = 512.0
NC = 2
NS = 16
NW = NC * NS
L = 16

_LEVELS = (
    (8.0, 64, 4096, 0, 512, 32),
    (16.0, 32, 1024, 4096, 128, 32),
    (32.0, 16, 256, 5120, 32, 32),
    (64.0, 8, 64, 5376, 16, 16),
    (128.0, 4, 16, 5440, 16, 4),
)
_MAX_K = 512
_TOTAL_ROWS = 5456

@functools.cache
def _build_sc_decode():
    mesh = plsc.VectorSubcoreMesh(
        core_axis_name="c", subcore_axis_name="s", num_cores=NC, num_subcores=NS
    )
    return functools.partial(
        pl.kernel,
        out_type=jax.ShapeDtypeStruct(
            (B, _TOTAL_ROWS, 4 + NUM_CLASSES), jnp.float32
        ),
        mesh=mesh,
        scratch_types=dict(
            cls_v=pltpu.VMEM((NUM_CLASSES, _MAX_K), jnp.float32),
            cntr_v=pltpu.VMEM((_MAX_K,), jnp.float32),
            regr_v=pltpu.VMEM((4, _MAX_K), jnp.float32),
            out_v=pltpu.VMEM((_MAX_K, 4 + NUM_CLASSES), jnp.float32),
        ),
        compiler_params=pltpu.CompilerParams(
            use_tc_tiling_on_sc=False, needs_layout_passes=False
        ),
    )(_sc_decode)


def _sc_decode(
    cls3, cntr3, regr3,
    cls4, cntr4, regr4,
    cls5, cntr5, regr5,
    cls6, cntr6, regr6,
    cls7, cntr7, regr7,
    out,
    *, cls_v, cntr_v, regr_v, out_v,
):
    wid = lax.axis_index("c") * NS + lax.axis_index("s")
    lane = lax.iota(jnp.int32, L)
    per_level = (
        (cls3, cntr3, regr3), (cls4, cntr4, regr4), (cls5, cntr5, regr5),
        (cls6, cntr6, regr6), (cls7, cntr7, regr7),
    )

    for (stride, w, hw, row_off, K, nw), (cls_h, cntr_h, regr_h) in zip(
        _LEVELS, per_level
    ):
        cpb = hw // K
        shift = w.bit_length() - 1

        def level_body(stride=stride, w=w, row_off=row_off, K=K, cpb=cpb,
                       shift=shift, cls_h=cls_h, cntr_h=cntr_h, regr_h=regr_h):
            b = wid // cpb
            off = (wid % cpb) * K
            pltpu.sync_copy(cls_h.at[b, :, pl.ds(off, K)], cls_v.at[:, pl.ds(0, K)])
            pltpu.sync_copy(cntr_h.at[b, pl.ds(off, K)], cntr_v.at[pl.ds(0, K)])
            pltpu.sync_copy(regr_h.at[b, :, pl.ds(off, K)], regr_v.at[:, pl.ds(0, K)])

            def group(g, carry):
                j0 = g * L
                rows = j0 + lane
                loc = off + rows
                xs = ((loc & (w - 1)).astype(jnp.float32) + 0.5) * stride
                ys = ((loc >> shift).astype(jnp.float32) + 0.5) * stride
                dl = jnp.exp(regr_v[0, pl.ds(j0, L)]) * stride
                dt = jnp.exp(regr_v[1, pl.ds(j0, L)]) * stride
                dr = jnp.exp(regr_v[2, pl.ds(j0, L)]) * stride
                db = jnp.exp(regr_v[3, pl.ds(j0, L)]) * stride
                xmin = jnp.minimum(jnp.maximum(xs - dl, 0.0), IMG)
                ymin = jnp.minimum(jnp.maximum(ys - dt, 0.0), IMG)
                xmax = jnp.minimum(jnp.maximum(xs + dr, 0.0), IMG)
                ymax = jnp.minimum(jnp.maximum(ys + db, 0.0), IMG)
                zero = lane * 0
                plsc.store_scatter(out_v, [rows, zero], xmin)
                plsc.store_scatter(out_v, [rows, zero + 1], ymin)
                plsc.store_scatter(out_v, [rows, zero + 2], xmax)
                plsc.store_scatter(out_v, [rows, zero + 3], ymax)
                cs = 1.0 / (1.0 + jnp.exp(-cntr_v[pl.ds(j0, L)]))

                def channel(c, carry2):
                    v = cls_v[c, pl.ds(j0, L)]
                    s = cs / (1.0 + jnp.exp(-v))
                    plsc.store_scatter(out_v, [rows, zero + (4 + c)], s)
                    return carry2

                return lax.fori_loop(0, NUM_CLASSES, channel, carry)

            lax.fori_loop(0, K // L, group, 0)
            pltpu.sync_copy(
                out_v.at[pl.ds(0, K), :], out.at[b, pl.ds(row_off + off, K), :]
            )

        if nw == NW:
            level_body()
        else:
            pl.when(wid < nw)(level_body)


def kernel(cls_p3, cntr_p3, regr_p3, cls_p4, cntr_p4, regr_p4,
           cls_p5, cntr_p5, regr_p5, cls_p6, cntr_p6, regr_p6,
           cls_p7, cntr_p7, regr_p7):
    args = []
    for cls, cntr, regr, (_, _, hw, _, _, _) in (
        (cls_p3, cntr_p3, regr_p3, _LEVELS[0]),
        (cls_p4, cntr_p4, regr_p4, _LEVELS[1]),
        (cls_p5, cntr_p5, regr_p5, _LEVELS[2]),
        (cls_p6, cntr_p6, regr_p6, _LEVELS[3]),
        (cls_p7, cntr_p7, regr_p7, _LEVELS[4]),
    ):
        args += [
            cls.reshape(B, NUM_CLASSES, hw),
            cntr.reshape(B, hw),
            regr.reshape(B, 4, hw),
        ]
    return _build_sc_decode()(*args)


# speedup vs baseline: 1.9173x; 1.5570x over previous
import functools

import jax
import jax.numpy as jnp
from jax import lax
from jax.experimental import pallas as pl
from jax.experimental.pallas import tpu as pltpu
from jax.experimental.pallas import tpu_sc as plsc

B = 4
NUM_CLASSES = 80
NCOLS = 4 + NUM_CLASSES
IMG = 512.0
NC = 2
NS = 16
NW = NC * NS
L = 16

_LEVELS = (
    (8.0, 64, 4096, 0, 512, 32),
    (16.0, 32, 1024, 4096, 128, 32),
    (32.0, 16, 256, 5120, 32, 32),
    (64.0, 8, 64, 5376, 16, 16),
    (128.0, 4, 16, 5440, 16, 4),
)
_MAX_K = 512
_TOTAL_ROWS = 5456


@functools.cache
def _build_sc_decode():
    mesh = plsc.VectorSubcoreMesh(
        core_axis_name="c", subcore_axis_name="s", num_cores=NC, num_subcores=NS
    )
    return functools.partial(
        pl.kernel,
        out_type=jax.ShapeDtypeStruct((B * _TOTAL_ROWS * NCOLS,), jnp.float32),
        mesh=mesh,
        scratch_types=dict(
            cls_v=pltpu.VMEM((NUM_CLASSES, _MAX_K), jnp.float32),
            cntr_v=pltpu.VMEM((_MAX_K,), jnp.float32),
            regr_v=pltpu.VMEM((4, _MAX_K), jnp.float32),
            out_v=pltpu.VMEM((_MAX_K * NCOLS,), jnp.float32),
        ),
        compiler_params=pltpu.CompilerParams(
            use_tc_tiling_on_sc=False, needs_layout_passes=False
        ),
    )(_sc_decode)


def _sc_decode(
    cls3, cntr3, regr3,
    cls4, cntr4, regr4,
    cls5, cntr5, regr5,
    cls6, cntr6, regr6,
    cls7, cntr7, regr7,
    out,
    *, cls_v, cntr_v, regr_v, out_v,
):
    wid = lax.axis_index("c") * NS + lax.axis_index("s")
    lane = lax.iota(jnp.int32, L)
    per_level = (
        (cls3, cntr3, regr3), (cls4, cntr4, regr4), (cls5, cntr5, regr5),
        (cls6, cntr6, regr6), (cls7, cntr7, regr7),
    )

    for (stride, w, hw, row_off, K, nw), (cls_h, cntr_h, regr_h) in zip(
        _LEVELS, per_level
    ):
        cpb = hw // K
        shift = w.bit_length() - 1

        def level_body(stride=stride, w=w, row_off=row_off, K=K, cpb=cpb,
                       shift=shift, cls_h=cls_h, cntr_h=cntr_h, regr_h=regr_h):
            b = wid // cpb
            off = (wid % cpb) * K
            pltpu.sync_copy(cls_h.at[b, :, pl.ds(off, K)], cls_v.at[:, pl.ds(0, K)])
            pltpu.sync_copy(cntr_h.at[b, pl.ds(off, K)], cntr_v.at[pl.ds(0, K)])
            pltpu.sync_copy(regr_h.at[b, :, pl.ds(off, K)], regr_v.at[:, pl.ds(0, K)])

            def group(g, carry):
                j0 = g * L
                rows = j0 + lane
                loc = off + rows
                xs = ((loc & (w - 1)).astype(jnp.float32) + 0.5) * stride
                ys = ((loc >> shift).astype(jnp.float32) + 0.5) * stride
                dl = jnp.exp(regr_v[0, pl.ds(j0, L)]) * stride
                dt = jnp.exp(regr_v[1, pl.ds(j0, L)]) * stride
                dr = jnp.exp(regr_v[2, pl.ds(j0, L)]) * stride
                db = jnp.exp(regr_v[3, pl.ds(j0, L)]) * stride
                xmin = jnp.minimum(jnp.maximum(xs - dl, 0.0), IMG)
                ymin = jnp.minimum(jnp.maximum(ys - dt, 0.0), IMG)
                xmax = jnp.minimum(jnp.maximum(xs + dr, 0.0), IMG)
                ymax = jnp.minimum(jnp.maximum(ys + db, 0.0), IMG)
                rbase = rows * NCOLS
                plsc.store_scatter(out_v, [rbase], xmin)
                plsc.store_scatter(out_v, [rbase + 1], ymin)
                plsc.store_scatter(out_v, [rbase + 2], xmax)
                plsc.store_scatter(out_v, [rbase + 3], ymax)
                cs = 1.0 / (1.0 + jnp.exp(-cntr_v[pl.ds(j0, L)]))
                sbase = rbase + 4

                @plsc.parallel_loop(0, NUM_CLASSES, unroll=8)
                def channel(c):
                    v = cls_v[c, pl.ds(j0, L)]
                    s = cs / (1.0 + jnp.exp(-v))
                    plsc.store_scatter(out_v, [sbase + c], s)

                return carry

            lax.fori_loop(0, K // L, group, 0)
            base = (b * _TOTAL_ROWS + row_off + off) * NCOLS
            pltpu.sync_copy(
                out_v.at[pl.ds(0, K * NCOLS)], out.at[pl.ds(base, K * NCOLS)]
            )

        if nw == NW:
            level_body()
        else:
            pl.when(wid < nw)(level_body)


def kernel(cls_p3, cntr_p3, regr_p3, cls_p4, cntr_p4, regr_p4,
           cls_p5, cntr_p5, regr_p5, cls_p6, cntr_p6, regr_p6,
           cls_p7, cntr_p7, regr_p7):
    args = []
    for cls, cntr, regr, (_, _, hw, _, _, _) in (
        (cls_p3, cntr_p3, regr_p3, _LEVELS[0]),
        (cls_p4, cntr_p4, regr_p4, _LEVELS[1]),
        (cls_p5, cntr_p5, regr_p5, _LEVELS[2]),
        (cls_p6, cntr_p6, regr_p6, _LEVELS[3]),
        (cls_p7, cntr_p7, regr_p7, _LEVELS[4]),
    ):
        args += [
            cls.reshape(B, NUM_CLASSES, hw),
            cntr.reshape(B, hw),
            regr.reshape(B, 4, hw),
        ]
    flat = _build_sc_decode()(*args)
    return flat.reshape(B, _TOTAL_ROWS, NCOLS)
